# baseline (speedup 1.0000x reference)
import functools

import jax
import jax.numpy as jnp
from jax import lax
from jax.experimental import pallas as pl
from jax.experimental.pallas import tpu as pltpu
from jax.experimental.pallas import tpu_sc as plsc

_LANES = 128
_SC_CHUNK = 256
_RED_Q = 4096


def _v_body(w1_ref, w2_ref, out_ref):
    v = jnp.dot(w1_ref[...], w2_ref[...], preferred_element_type=jnp.float32)
    out_ref[...] = v[:, 0]


def _tail_body(tab_ref, w1_ref, w2_ref, out_ref):
    v = jnp.dot(w1_ref[...], w2_ref[...], preferred_element_type=jnp.float32)
    acc = lax.dot_general(v.T, tab_ref[...], (((1,), (1,)), ((), ())),
                          preferred_element_type=jnp.float32)
    out_ref[...] = acc[0]


def _reduce_body(part_ref, out_ref):
    lanes = lax.broadcasted_iota(jnp.int32, (128, 8), 0)
    cols = lax.broadcasted_iota(jnp.int32, (128, 8), 1)
    sel = (lanes // 16 == cols).astype(jnp.float32)
    mm = jnp.dot(part_ref[...], sel, preferred_element_type=jnp.float32)
    out_ref[...] = mm.T


def _head_body(t_ref, w3_ref, b1_ref, w2_ref, b2_ref, b3_ref, out_ref):
    c = jnp.dot(b1_ref[...], w2_ref[...], preferred_element_type=jnp.float32)
    const = (c[0, 0] + b2_ref[0, 0]) * jnp.sum(w3_ref[...]) + b3_ref[0, 0]
    acc = jnp.dot(t_ref[...], w3_ref[...], preferred_element_type=jnp.float32)
    out_ref[...] = jax.nn.sigmoid(acc + const)


def _make_partial(num_workers, rows_per_w, dim):
    nc = plsc.get_sparse_core_info().num_cores
    mesh = plsc.VectorSubcoreMesh(core_axis_name="c", subcore_axis_name="s")
    n_chunks = rows_per_w // _SC_CHUNK
    pb = _SC_CHUNK * 16

    @functools.partial(
        pl.kernel,
        mesh=mesh,
        out_type=jax.ShapeDtypeStruct(
            (num_workers * rows_per_w * 16 // 128, 128), jnp.float32),
        scratch_types=[
            pltpu.VMEM((_SC_CHUNK, dim), jnp.float32),
            pltpu.VMEM((_SC_CHUNK, dim), jnp.float32),
            pltpu.VMEM((pb // 128, 128), jnp.float32),
            pltpu.VMEM((pb // 128, 128), jnp.float32),
            pltpu.VMEM((dim,), jnp.float32),
            pltpu.SemaphoreType.DMA,
            pltpu.SemaphoreType.DMA,
            pltpu.SemaphoreType.DMA,
            pltpu.SemaphoreType.DMA,
        ],
    )
    def partial_kernel(table_hbm, v_hbm, out_hbm,
                       rows_a, rows_b, pk_a, pk_b, v_v,
                       sem0, sem1, semp0, semp1):
        w = lax.axis_index("s") * nc + lax.axis_index("c")
        base = w * rows_per_w
        pltpu.sync_copy(v_hbm, v_v)
        v0 = v_v[pl.ds(0, 16)]
        v1 = v_v[pl.ds(16, 16)]
        v2 = v_v[pl.ds(32, 16)]
        v3 = v_v[pl.ds(48, 16)]

        def fire(ch, buf_ref, sem):
            r0 = pl.multiple_of(base + ch * _SC_CHUNK, 8)
            pltpu.async_copy(
                table_hbm.at[pl.ds(r0, _SC_CHUNK), :], buf_ref, sem)

        def drain(buf_ref, sem):
            pltpu.make_async_copy(
                table_hbm.at[pl.ds(0, _SC_CHUNK), :], buf_ref, sem).wait()

        def compute(buf_ref, pk_ref):
            def group(g, carry):
                for r in range(16):
                    row = g * 16 + r
                    s = (buf_ref[row, pl.ds(0, 16)] * v0
                         + buf_ref[row, pl.ds(16, 16)] * v1
                         + buf_ref[row, pl.ds(32, 16)] * v2
                         + buf_ref[row, pl.ds(48, 16)] * v3)
                    pk_ref[g * 2 + r // 8, pl.ds((r % 8) * 16, 16)] = s
                return carry

            lax.fori_loop(0, _SC_CHUNK // 16, group, 0, unroll=False)

        def flush(ch, pk_ref, semp):
            row0 = pl.multiple_of((base + ch * _SC_CHUNK) // 8, 8)
            pltpu.async_copy(
                pk_ref, out_hbm.at[pl.ds(row0, _SC_CHUNK // 8), :], semp)

        def flush_wait(pk_ref, semp):
            pltpu.make_async_copy(
                pk_ref, out_hbm.at[pl.ds(0, _SC_CHUNK // 8), :], semp).wait()

        fire(0, rows_a, sem0)

        def pair(it, carry):
            c0 = it * 2
            fire(c0 + 1, rows_b, sem1)
            drain(rows_a, sem0)

            @pl.when(it > 0)
            def _():
                flush_wait(pk_a, semp0)

            compute(rows_a, pk_a)
            flush(c0, pk_a, semp0)

            @pl.when(c0 + 2 < n_chunks)
            def _():
                fire(c0 + 2, rows_a, sem0)

            drain(rows_b, sem1)

            @pl.when(it > 0)
            def _():
                flush_wait(pk_b, semp1)

            compute(rows_b, pk_b)
            flush(c0 + 1, pk_b, semp1)
            return carry

        lax.fori_loop(0, n_chunks // 2, pair, 0, unroll=False)
        flush_wait(pk_a, semp0)
        flush_wait(pk_b, semp1)

    return partial_kernel


def _make_gather(num_workers, rows, sc_rows, n_part):
    nc = plsc.get_sparse_core_info().num_cores
    mesh = plsc.VectorSubcoreMesh(core_axis_name="c", subcore_axis_name="s")

    @functools.partial(
        pl.kernel,
        mesh=mesh,
        out_type=jax.ShapeDtypeStruct((num_workers, rows, _LANES), jnp.float32),
        scratch_types=[
            pltpu.VMEM((rows, _LANES), jnp.int32),
            pltpu.VMEM((rows, _LANES), jnp.float32),
            pltpu.SemaphoreType.DMA,
        ],
    )
    def gather_kernel(ids_hbm, p_hbm, out_hbm, idx_v, val_v, sem):
        wid = lax.axis_index("s") * nc + lax.axis_index("c")
        pltpu.sync_copy(ids_hbm.at[wid], idx_v)

        def xform(j, carry):
            for l in range(8):
                sl = idx_v[j, pl.ds(l * 16, 16)]
                q = jnp.right_shift(sl, 3)
                jj = jnp.bitwise_and(sl, 7)
                idx_v[j, pl.ds(l * 16, 16)] = jnp.where(
                    sl < sc_rows, jj * n_part + q, sl)
            return carry

        lax.fori_loop(0, rows, xform, 0, unroll=False)

        def fire(j, carry):
            pltpu.async_copy(p_hbm.at[idx_v.at[j]], val_v.at[j], sem)
            return carry

        lax.fori_loop(0, rows, fire, 0, unroll=False)

        def drain(j, carry):
            pltpu.make_async_copy(p_hbm.at[idx_v.at[j]], val_v.at[j], sem).wait()
            return carry

        lax.fori_loop(0, rows, drain, 0, unroll=False)
        pltpu.sync_copy(val_v, out_hbm.at[wid])

    return gather_kernel


def kernel(input_ids, emb_table, W1, b1, W2, b2, W3, b3):
    B, S = input_ids.shape
    V, D = emb_table.shape
    H = W1.shape[1]

    info = plsc.get_sparse_core_info()
    nw = info.num_cores * info.num_subcores

    v = pl.pallas_call(
        _v_body,
        in_specs=[
            pl.BlockSpec((D, H), lambda: (0, 0)),
            pl.BlockSpec((H, 1), lambda: (0, 0)),
        ],
        out_specs=pl.BlockSpec((D,), lambda: (0,)),
        out_shape=jax.ShapeDtypeStruct((D,), jnp.float32),
    )(W1, W2)

    rows_per_w = (V // (nw * _SC_CHUNK)) * _SC_CHUNK
    sc_rows = rows_per_w * nw
    part = _make_partial(nw, rows_per_w, D)(emb_table, v)

    n_part = sc_rows * 16 // 128
    red_grid = (n_part + _RED_Q - 1) // _RED_Q
    p_main = pl.pallas_call(
        _reduce_body,
        grid=(red_grid,),
        in_specs=[pl.BlockSpec((_RED_Q, 128), lambda i: (i, 0))],
        out_specs=pl.BlockSpec((8, _RED_Q), lambda i: (0, i)),
        out_shape=jax.ShapeDtypeStruct((8, n_part), jnp.float32),
    )(part).reshape(sc_rows)

    tail_blk = 8000
    assert V % tail_blk == 0
    p_tail_full = pl.pallas_call(
        _tail_body,
        grid=(1,),
        in_specs=[
            pl.BlockSpec((tail_blk, D), lambda i: (V // tail_blk - 1, 0)),
            pl.BlockSpec((D, H), lambda i: (0, 0)),
            pl.BlockSpec((H, 1), lambda i: (0, 0)),
        ],
        out_specs=pl.BlockSpec((tail_blk,), lambda i: (0,)),
        out_shape=jax.ShapeDtypeStruct((tail_blk,), jnp.float32),
    )(emb_table, W1, W2)
    p = jnp.concatenate([p_main, p_tail_full[tail_blk - (V - sc_rows):]])

    total = B * S
    rows = total // (nw * _LANES)
    ids3 = input_ids.reshape(nw, rows, _LANES)
    t = _make_gather(nw, rows, sc_rows, n_part)(ids3, p)
    t = t.reshape(B, S)

    out = pl.pallas_call(
        _head_body,
        in_specs=[
            pl.BlockSpec((B, S), lambda: (0, 0)),
            pl.BlockSpec((S, 1), lambda: (0, 0)),
            pl.BlockSpec((1, H), lambda: (0, 0)),
            pl.BlockSpec((H, 1), lambda: (0, 0)),
            pl.BlockSpec((1, 1), lambda: (0, 0)),
            pl.BlockSpec((1, 1), lambda: (0, 0)),
        ],
        out_specs=pl.BlockSpec((B, 1), lambda: (0, 0)),
        out_shape=jax.ShapeDtypeStruct((B, 1), jnp.float32),
    )(t, W3, b1.reshape(1, H), W2, b2.reshape(1, 1), b3.reshape(1, 1))
    return out
